# baseline (speedup 1.0000x reference)
import functools

import jax
import jax.numpy as jnp
from jax import lax
from jax.experimental import pallas as pl
from jax.experimental.pallas import tpu as pltpu
from jax.experimental.pallas import tpu_sc as plsc

_N = 10000
_BR = 128
_NB = 79

_L = 16
_EPAD = 172032
_ESC = _EPAD // 2
_SPAN = _ESC // 16
_NCH = _SPAN // 128
_NCP = 48
_AR = 10112
_DUM = 10048
_RT = _AR // 16


def _zero_acc(rowbuf, acc, r0):
    def zrow(i, _):
        for kk in range(8):
            rowbuf[i, pl.ds(kk * _L, _L)] = jnp.zeros((_L,), jnp.float32)
        return 0
    lax.fori_loop(0, 128, zrow, 0)
    for q in range(4):
        pltpu.sync_copy(rowbuf, acc.at[pl.ds(r0 + q * 128, 128)])
    pltpu.sync_copy(rowbuf.at[pl.ds(0, _RT - 4 * 128)],
                    acc.at[pl.ds(r0 + 4 * 128, _RT - 4 * 128)])


def _make_deg_kernel():
    mesh = plsc.VectorSubcoreMesh(core_axis_name="c", subcore_axis_name="s")
    scratch = [
        pltpu.VMEM((_NCP, 128), jnp.int32),
        pltpu.VMEM((128, 128), jnp.float32),
        pltpu.VMEM_SHARED((_AR, 128), jnp.float32),
        pltpu.SemaphoreType.DMA,
        pltpu.SemaphoreType.DMA,
    ]

    def body(sct2_h, out_h, sct_all, rowbuf, acc, sw0, sw1):
        c = lax.axis_index("c")
        s = lax.axis_index("s")
        cbase = (c * 16 + s) * _NCP
        r0 = s * _RT
        pltpu.sync_copy(sct2_h.at[pl.ds(cbase, _NCP)], sct_all)
        _zero_acc(rowbuf, acc, r0)

        def orow(i, _):
            for kk in range(8):
                rowbuf[i, pl.ds(kk * _L, _L)] = jnp.ones((_L,), jnp.float32)
            return 0
        lax.fori_loop(0, 128, orow, 0)
        plsc.subcore_barrier()

        def w_dst(j):
            return acc.at[sct_all.at[j]]

        pltpu.async_copy(rowbuf, w_dst(0), sw0, add=True)
        pltpu.async_copy(rowbuf, w_dst(1), sw1, add=True)

        def pair(jj, _):
            j0 = jj * 2
            pltpu.make_async_copy(rowbuf, w_dst(j0), sw0).wait()
            pltpu.async_copy(rowbuf, w_dst(j0 + 2), sw0, add=True)
            pltpu.make_async_copy(rowbuf, w_dst(j0 + 1), sw1).wait()
            pltpu.async_copy(rowbuf, w_dst(j0 + 3), sw1, add=True)
            return 0
        lax.fori_loop(0, _NCH // 2 - 1, pair, 0)
        pltpu.make_async_copy(rowbuf, w_dst(_NCH - 2), sw0).wait()
        pltpu.make_async_copy(rowbuf, w_dst(_NCH - 1), sw1).wait()
        plsc.subcore_barrier()
        pltpu.sync_copy(acc.at[pl.ds(r0, _RT)],
                        out_h.at[pl.ds(c * _AR + r0, _RT)])

    return pl.kernel(
        body,
        out_type=jax.ShapeDtypeStruct((2 * _AR, 128), jnp.float32),
        mesh=mesh, scratch_types=scratch)


def _stage_all(src2_h, sct2_h, cbase, sidx_all, sct_all):
    pltpu.sync_copy(src2_h.at[pl.ds(cbase, _NCP)], sidx_all)
    pltpu.sync_copy(sct2_h.at[pl.ds(cbase, _NCP)], sct_all)


def _make_gcn_kernel(C):
    ncol = C // 128
    mesh = plsc.VectorSubcoreMesh(core_axis_name="c", subcore_axis_name="s")
    scratch = [
        pltpu.VMEM((_NCP, 128), jnp.int32),
        pltpu.VMEM((_NCP, 128), jnp.int32),
        pltpu.VMEM((128, 128), jnp.float32),
        pltpu.VMEM((128, 128), jnp.float32),
        pltpu.VMEM_SHARED((_AR, 128), jnp.float32),
        pltpu.SemaphoreType.DMA,
        pltpu.SemaphoreType.DMA,
        pltpu.SemaphoreType.DMA,
        pltpu.SemaphoreType.DMA,
    ]

    def body(src2_h, sct2_h, tab_h, out_h,
             sidx_all, sct_all, rb0, rb1, acc, sg0, sg1, sw0, sw1):
        c = lax.axis_index("c")
        s = lax.axis_index("s")
        cbase = (c * 16 + s) * _NCP
        nch = _NCH
        r0 = s * _RT
        _stage_all(src2_h, sct2_h, cbase, sidx_all, sct_all)

        for k in range(ncol):
            _zero_acc(rb0, acc, r0)
            plsc.subcore_barrier()

            def g_hi(j):
                return tab_h.at[sidx_all.at[j, pl.ds(0, 64)],
                                pl.ds(k * 128, 128)]

            def g_lo(j):
                return tab_h.at[sidx_all.at[j, pl.ds(64, 64)],
                                pl.ds(k * 128, 128)]

            def fire_g(j, rb, sg):
                pltpu.async_copy(g_hi(j), rb.at[pl.ds(0, 64)], sg)
                pltpu.async_copy(g_lo(j), rb.at[pl.ds(64, 64)], sg)

            def wait_g(j, rb, sg):
                pltpu.make_async_copy(g_hi(j), rb.at[pl.ds(0, 64)], sg).wait()
                pltpu.make_async_copy(g_lo(j), rb.at[pl.ds(64, 64)], sg).wait()

            def w_dst(j):
                return acc.at[sct_all.at[j]]

            fire_g(0, rb0, sg0)
            fire_g(1, rb1, sg1)

            def pair(jj, _):
                j0 = jj * 2
                wait_g(j0, rb0, sg0)
                pltpu.async_copy(rb0, w_dst(j0), sw0, add=True)
                wait_g(j0 + 1, rb1, sg1)
                pltpu.async_copy(rb1, w_dst(j0 + 1), sw1, add=True)
                pltpu.make_async_copy(rb0, w_dst(j0), sw0).wait()
                fire_g(j0 + 2, rb0, sg0)
                pltpu.make_async_copy(rb1, w_dst(j0 + 1), sw1).wait()
                fire_g(j0 + 3, rb1, sg1)
                return 0
            lax.fori_loop(0, nch // 2 - 1, pair, 0)
            jl = nch - 2
            wait_g(jl, rb0, sg0)
            pltpu.async_copy(rb0, w_dst(jl), sw0, add=True)
            wait_g(jl + 1, rb1, sg1)
            pltpu.async_copy(rb1, w_dst(jl + 1), sw1, add=True)
            pltpu.make_async_copy(rb0, w_dst(jl), sw0).wait()
            pltpu.make_async_copy(rb1, w_dst(jl + 1), sw1).wait()

            plsc.subcore_barrier()
            pltpu.sync_copy(acc.at[pl.ds(r0, _RT)],
                            out_h.at[pl.ds(c * _AR + r0, _RT),
                                     pl.ds(k * 128, 128)])
            plsc.subcore_barrier()

    return pl.kernel(
        body,
        out_type=jax.ShapeDtypeStruct((2 * _AR, C), jnp.float32),
        mesh=mesh, scratch_types=scratch)


def _make_gat_kernel(C):
    ncol = C // 128
    mesh = plsc.VectorSubcoreMesh(core_axis_name="c", subcore_axis_name="s")
    scratch = [
        pltpu.VMEM((_NCP, 128), jnp.int32),
        pltpu.VMEM((_NCP, 128), jnp.int32),
        pltpu.VMEM((128,), jnp.int32),
        pltpu.VMEM((128, 128), jnp.float32),
        pltpu.VMEM((128, 128), jnp.float32),
        pltpu.VMEM((2048,), jnp.float32),
        pltpu.VMEM((2048,), jnp.float32),
        pltpu.VMEM_SHARED((_AR, 128), jnp.float32),
        pltpu.SemaphoreType.DMA,
        pltpu.SemaphoreType.DMA,
        pltpu.SemaphoreType.DMA,
        pltpu.SemaphoreType.DMA,
    ]

    def body(src2_h, sct2_h, tab_h, srep_h, drep_h, out_h, w_h,
             sidx_all, sct_all, didx, rb0, rb1, wb0, wb1, acc,
             sg0, sg1, sw0, sw1):
        c = lax.axis_index("c")
        s = lax.axis_index("s")
        cbase = (c * 16 + s) * _NCP
        nch = _NCH
        ebase = (c * 16 + s) * _SPAN
        r0 = s * _RT
        _stage_all(src2_h, sct2_h, cbase, sidx_all, sct_all)

        _zero_acc(rb0, acc, r0)
        plsc.subcore_barrier()

        def wsl(j):
            return w_h.at[pl.ds((ebase + j * 128) * _L, 128 * _L)]

        def chunk0(j, _):
            pltpu.async_copy(srep_h.at[sidx_all.at[j]], rb0, sg0)
            for kk in range(8):
                v = sct_all[j, pl.ds(kk * _L, _L)]
                didx[pl.ds(kk * _L, _L)] = jnp.where(v == _DUM, 0, v)
            pl.when(j > 0)(lambda: pltpu.make_async_copy(
                rb1, acc.at[sct_all.at[j - 1]], sw1).wait())
            pltpu.async_copy(drep_h.at[didx], rb1, sg1)
            pl.when(j > 0)(lambda: pltpu.make_async_copy(
                wb0, wsl(j - 1), sw0).wait())
            pltpu.make_async_copy(srep_h.at[sidx_all.at[j]], rb0, sg0).wait()
            pltpu.make_async_copy(drep_h.at[didx], rb1, sg1).wait()

            def wrow(i, _):
                pv = rb0[i, pl.ds(0, _L)]
                rv = rb0[i, pl.ds(_L, _L)]
                qv = rb1[i, pl.ds(0, _L)]
                tv = rb1[i, pl.ds(_L, _L)]
                wv = jnp.maximum(pv * qv, rv * tv)
                wb0[pl.ds(i * _L, _L)] = wv
                for kk in range(8):
                    rb1[i, pl.ds(kk * _L, _L)] = wv
                return 0
            lax.fori_loop(0, 128, wrow, 0)
            pltpu.async_copy(wb0, wsl(j), sw0)
            pltpu.async_copy(rb1, acc.at[sct_all.at[j]], sw1, add=True)
            return 0
        lax.fori_loop(0, nch, chunk0, 0)
        pltpu.make_async_copy(rb1, acc.at[sct_all.at[nch - 1]], sw1).wait()
        pltpu.make_async_copy(wb0, wsl(nch - 1), sw0).wait()
        plsc.subcore_barrier()
        pltpu.sync_copy(acc.at[pl.ds(r0, _RT)],
                        out_h.at[pl.ds(c * _AR + r0, _RT),
                                 pl.ds(C, 128)])
        plsc.subcore_barrier()

        for k in range(ncol):
            _zero_acc(rb0, acc, r0)
            plsc.subcore_barrier()

            def g_ref(j):
                return tab_h.at[sidx_all.at[j], pl.ds(k * 128, 128)]

            def u_ref(j):
                return w_h.at[pl.ds((ebase + j * 128) * _L, 128 * _L)]

            def w_dst(j):
                return acc.at[sct_all.at[j]]

            def fire_g(j, rb, wb, sg):
                pltpu.async_copy(g_ref(j), rb, sg)
                pltpu.async_copy(u_ref(j), wb, sg)

            def wait_g(j, rb, wb, sg):
                pltpu.make_async_copy(g_ref(j), rb, sg).wait()
                pltpu.make_async_copy(u_ref(j), wb, sg).wait()

            def scale(rb, wb):
                def srow(i, _):
                    wv = wb[pl.ds(i * _L, _L)]
                    for kk in range(8):
                        rb[i, pl.ds(kk * _L, _L)] = (
                            rb[i, pl.ds(kk * _L, _L)] * wv)
                    return 0
                lax.fori_loop(0, 128, srow, 0)

            fire_g(0, rb0, wb0, sg0)
            fire_g(1, rb1, wb1, sg1)

            def pair(jj, _):
                j0 = jj * 2
                wait_g(j0, rb0, wb0, sg0)
                scale(rb0, wb0)
                pltpu.async_copy(rb0, w_dst(j0), sw0, add=True)
                wait_g(j0 + 1, rb1, wb1, sg1)
                scale(rb1, wb1)
                pltpu.async_copy(rb1, w_dst(j0 + 1), sw1, add=True)
                pltpu.make_async_copy(rb0, w_dst(j0), sw0).wait()
                fire_g(j0 + 2, rb0, wb0, sg0)
                pltpu.make_async_copy(rb1, w_dst(j0 + 1), sw1).wait()
                fire_g(j0 + 3, rb1, wb1, sg1)
                return 0
            lax.fori_loop(0, nch // 2 - 1, pair, 0)
            jl = nch - 2
            wait_g(jl, rb0, wb0, sg0)
            scale(rb0, wb0)
            pltpu.async_copy(rb0, w_dst(jl), sw0, add=True)
            wait_g(jl + 1, rb1, wb1, sg1)
            scale(rb1, wb1)
            pltpu.async_copy(rb1, w_dst(jl + 1), sw1, add=True)
            pltpu.make_async_copy(rb0, w_dst(jl), sw0).wait()
            pltpu.make_async_copy(rb1, w_dst(jl + 1), sw1).wait()

            plsc.subcore_barrier()
            pltpu.sync_copy(acc.at[pl.ds(r0, _RT)],
                            out_h.at[pl.ds(c * _AR + r0, _RT),
                                     pl.ds(k * 128, 128)])
            plsc.subcore_barrier()

    return pl.kernel(
        body,
        out_type=(jax.ShapeDtypeStruct((2 * _AR, C + 128), jnp.float32),
                  jax.ShapeDtypeStruct((_EPAD * _L,), jnp.float32)),
        mesh=mesh, scratch_types=scratch)


def _splat(p, r):
    br = p.shape[0]
    return jnp.concatenate(
        [jnp.tile(p, (1, _L)), jnp.tile(r, (1, _L)),
         jnp.zeros((br, 128 - 2 * _L), jnp.float32)], axis=1)


def _t1_kern(x_ref, w_ref, dga_ref, dgb_ref, o_ref):
    deg = dga_ref[...][:, :1] + dgb_ref[...][:, :1]
    acc = jnp.dot(x_ref[...], w_ref[...], preferred_element_type=jnp.float32)
    o_ref[...] = acc * lax.rsqrt(deg)


def _t2_kern(ma_ref, mb_ref, dga_ref, dgb_ref, b_ref, w_ref, a_ref,
             h_ref, t_ref, s_ref, d_ref):
    deg = dga_ref[...][:, :1] + dgb_ref[...][:, :1]
    h = (ma_ref[...] + mb_ref[...]) * lax.rsqrt(deg) + b_ref[...]
    h_ref[...] = h
    xw = jnp.dot(h, w_ref[...], preferred_element_type=jnp.float32)
    t_ref[...] = xw
    sv = jnp.dot(xw, a_ref[...], preferred_element_type=jnp.float32)
    s1 = sv[:, 0:1]
    s2 = sv[:, 1:2]
    s_ref[...] = _splat(jnp.exp(s1), jnp.exp(0.2 * s1))
    d_ref[...] = _splat(jnp.exp(s2), jnp.exp(0.2 * s2))


def _t3_kern(h_ref, na_ref, nb_ref, batt_ref, g_ref, be_ref, wl_ref, bl_ref,
             watt_ref, a_ref, h2_ref, t_ref, s_ref, d_ref, *, d):
    nd = na_ref[...] + nb_ref[...]
    a = nd[:, :d] / nd[:, d:d + 1] + batt_ref[...]
    z = h_ref[...] + a
    mu = jnp.mean(z, axis=-1, keepdims=True)
    var = jnp.mean((z - mu) ** 2, axis=-1, keepdims=True)
    zn = (z - mu) * lax.rsqrt(var + 1e-5) * g_ref[...] + be_ref[...]
    h2 = jnp.maximum(
        jnp.dot(zn, wl_ref[...], preferred_element_type=jnp.float32)
        + bl_ref[...], 0.0)
    h2_ref[...] = h2
    xw = jnp.dot(h2, watt_ref[...], preferred_element_type=jnp.float32)
    t_ref[...] = xw
    sv = jnp.dot(xw, a_ref[...], preferred_element_type=jnp.float32)
    s1 = sv[:, 0:1]
    s2 = sv[:, 1:2]
    s_ref[...] = _splat(jnp.exp(s1), jnp.exp(0.2 * s1))
    d_ref[...] = _splat(jnp.exp(s2), jnp.exp(0.2 * s2))


def _t4_kern(h_ref, na_ref, nb_ref, batt_ref, g_ref, be_ref, wl_ref, bl_ref,
             wconv_ref, dga_ref, dgb_ref, t_ref, *, d):
    nd = na_ref[...] + nb_ref[...]
    a = nd[:, :d] / nd[:, d:d + 1] + batt_ref[...]
    z = h_ref[...] + a
    mu = jnp.mean(z, axis=-1, keepdims=True)
    var = jnp.mean((z - mu) ** 2, axis=-1, keepdims=True)
    zn = (z - mu) * lax.rsqrt(var + 1e-5) * g_ref[...] + be_ref[...]
    h3 = jnp.maximum(
        jnp.dot(zn, wl_ref[...], preferred_element_type=jnp.float32)
        + bl_ref[...], 0.0)
    deg = dga_ref[...][:, :1] + dgb_ref[...][:, :1]
    xw = jnp.dot(h3, wconv_ref[...], preferred_element_type=jnp.float32)
    br = xw.shape[0]
    t_ref[...] = jnp.concatenate(
        [xw * lax.rsqrt(deg), jnp.zeros((br, 64), jnp.float32)], axis=1)


def _t5_kern(ma_ref, mb_ref, dga_ref, dgb_ref, bconv_ref,
             w1_ref, b1_ref, w2_ref, b2_ref, o_ref):
    deg = dga_ref[...][:, :1] + dgb_ref[...][:, :1]
    m = ma_ref[...][:, :64] + mb_ref[...][:, :64]
    h = m * lax.rsqrt(deg) + bconv_ref[...]
    h = jnp.maximum(
        jnp.dot(h, w1_ref[...], preferred_element_type=jnp.float32)
        + b1_ref[...], 0.0)
    z = jnp.dot(h, w2_ref[...], preferred_element_type=jnp.float32) + b2_ref[...]
    lse = jax.scipy.special.logsumexp(z, axis=-1, keepdims=True)
    o_ref[...] = z - lse


def _rowA(c):
    return pl.BlockSpec((_BR, c), lambda i: (i, 0))


def _rowB(c):
    return pl.BlockSpec((_BR, c), lambda i: (i + _AR // _BR, 0))


def _full(r, c):
    return pl.BlockSpec((r, c), lambda i: (0, 0))


def _call_rows(kern, ins, in_specs, out_cols):
    out_specs = [_rowA(c) for c in out_cols]
    out_shape = [jax.ShapeDtypeStruct((_N, c), jnp.float32) for c in out_cols]
    if len(out_cols) == 1:
        out_specs, out_shape = out_specs[0], out_shape[0]
    return pl.pallas_call(
        kern, grid=(_NB,), in_specs=in_specs,
        out_specs=out_specs, out_shape=out_shape)(*ins)


def kernel(x, edge_index, params):
    p = params
    ar = jnp.arange(_N, dtype=edge_index.dtype)
    npadE = _EPAD - 17 * _N
    srcp = jnp.concatenate(
        [edge_index[0], ar, jnp.zeros((npadE,), jnp.int32)])
    dstp = jnp.concatenate(
        [edge_index[1], ar, jnp.full((npadE,), 1 << 29, jnp.int32)])
    sctp = jnp.where(dstp < _N, dstp, _DUM)
    srcp = jnp.pad(srcp.reshape(32, _NCH, 128),
                   ((0, 0), (0, _NCP - _NCH), (0, 0))).reshape(32 * _NCP, 128)
    sctp = jnp.pad(sctp.reshape(32, _NCH, 128),
                   ((0, 0), (0, _NCP - _NCH), (0, 0)),
                   constant_values=_DUM).reshape(32 * _NCP, 128)

    deg = _make_deg_kernel()(sctp)

    tab1 = _call_rows(
        _t1_kern, (x, p['W_emb'], deg, deg),
        [_rowA(256), _full(256, 512), _rowA(128), _rowB(128)], [512])
    m1 = _make_gcn_kernel(512)(srcp, sctp, tab1)

    a1 = jnp.stack([p['a1_src'], p['a1_dst']], axis=1)
    h, tab2, srep1, drep1 = _call_rows(
        _t2_kern,
        (m1, m1, deg, deg, p['b_emb'].reshape(1, -1), p['W_att1'], a1),
        [_rowA(512), _rowB(512), _rowA(128), _rowB(128),
         _full(1, 512), _full(512, 512), _full(512, 2)],
        [512, 512, 128, 128])
    nd1, _w1 = _make_gat_kernel(512)(srcp, sctp, tab2, srep1, drep1)

    a2 = jnp.stack([p['a2_src'], p['a2_dst']], axis=1)
    h2, tab3, srep2, drep2 = _call_rows(
        functools.partial(_t3_kern, d=512),
        (h, nd1, nd1, p['b_att1'].reshape(1, -1), p['g_n1'].reshape(1, -1),
         p['be_n1'].reshape(1, -1), p['W_l1'], p['b_l1'].reshape(1, -1),
         p['W_att2'], a2),
        [_rowA(512), _rowA(640), _rowB(640), _full(1, 512), _full(1, 512),
         _full(1, 512), _full(512, 256), _full(1, 256), _full(256, 256),
         _full(256, 2)],
        [256, 256, 128, 128])
    nd2, _w2 = _make_gat_kernel(256)(srcp, sctp, tab3, srep2, drep2)

    tab4 = _call_rows(
        functools.partial(_t4_kern, d=256),
        (h2, nd2, nd2, p['b_att2'].reshape(1, -1), p['g_n2'].reshape(1, -1),
         p['be_n2'].reshape(1, -1), p['W_l2'], p['b_l2'].reshape(1, -1),
         p['W_conv'], deg, deg),
        [_rowA(256), _rowA(384), _rowB(384), _full(1, 256), _full(1, 256),
         _full(1, 256), _full(256, 128), _full(1, 128), _full(128, 64),
         _rowA(128), _rowB(128)],
        [128])
    m3 = _make_gcn_kernel(128)(srcp, sctp, tab4)

    out = _call_rows(
        _t5_kern,
        (m3, m3, deg, deg, p['b_conv'].reshape(1, -1), p['W_c1'],
         p['b_c1'].reshape(1, -1), p['W_c2'], p['b_c2'].reshape(1, -1)),
        [_rowA(128), _rowB(128), _rowA(128), _rowB(128), _full(1, 64),
         _full(64, 32), _full(1, 32), _full(32, 10), _full(1, 10)],
        [10])
    return out


# speedup vs baseline: 1.0071x; 1.0071x over previous
import functools

import jax
import jax.numpy as jnp
from jax import lax
from jax.experimental import pallas as pl
from jax.experimental.pallas import tpu as pltpu
from jax.experimental.pallas import tpu_sc as plsc

_N = 10000
_BR = 128
_NB = 79

_L = 16
_EPAD = 172032
_ESC = _EPAD // 2
_SPAN = _ESC // 16
_NCH = _SPAN // 128
_NCP = 48
_AR = 10112
_DUM = 10048
_RT = _AR // 16


def _zero_acc(rowbuf, acc, r0):
    def zrow(i, _):
        for kk in range(8):
            rowbuf[i, pl.ds(kk * _L, _L)] = jnp.zeros((_L,), jnp.float32)
        return 0
    lax.fori_loop(0, 128, zrow, 0)
    for q in range(4):
        pltpu.sync_copy(rowbuf, acc.at[pl.ds(r0 + q * 128, 128)])
    pltpu.sync_copy(rowbuf.at[pl.ds(0, _RT - 4 * 128)],
                    acc.at[pl.ds(r0 + 4 * 128, _RT - 4 * 128)])


def _make_deg_kernel():
    mesh = plsc.VectorSubcoreMesh(core_axis_name="c", subcore_axis_name="s")
    scratch = [
        pltpu.VMEM((_NCP, 128), jnp.int32),
        pltpu.VMEM((128, 128), jnp.float32),
        pltpu.VMEM_SHARED((_AR, 128), jnp.float32),
        pltpu.SemaphoreType.DMA,
        pltpu.SemaphoreType.DMA,
    ]

    def body(sct2_h, out_h, sct_all, rowbuf, acc, sw0, sw1):
        c = lax.axis_index("c")
        s = lax.axis_index("s")
        cbase = (c * 16 + s) * _NCP
        r0 = s * _RT
        pltpu.sync_copy(sct2_h.at[pl.ds(cbase, _NCP)], sct_all)
        _zero_acc(rowbuf, acc, r0)

        def orow(i, _):
            for kk in range(8):
                rowbuf[i, pl.ds(kk * _L, _L)] = jnp.ones((_L,), jnp.float32)
            return 0
        lax.fori_loop(0, 128, orow, 0)
        plsc.subcore_barrier()

        def w_dst(j):
            return acc.at[sct_all.at[j]]

        pltpu.async_copy(rowbuf, w_dst(0), sw0, add=True)
        pltpu.async_copy(rowbuf, w_dst(1), sw1, add=True)

        def pair(jj, _):
            j0 = jj * 2
            pltpu.make_async_copy(rowbuf, w_dst(j0), sw0).wait()
            pltpu.async_copy(rowbuf, w_dst(j0 + 2), sw0, add=True)
            pltpu.make_async_copy(rowbuf, w_dst(j0 + 1), sw1).wait()
            pltpu.async_copy(rowbuf, w_dst(j0 + 3), sw1, add=True)
            return 0
        lax.fori_loop(0, _NCH // 2 - 1, pair, 0)
        pltpu.make_async_copy(rowbuf, w_dst(_NCH - 2), sw0).wait()
        pltpu.make_async_copy(rowbuf, w_dst(_NCH - 1), sw1).wait()
        plsc.subcore_barrier()
        pltpu.sync_copy(acc.at[pl.ds(r0, _RT)],
                        out_h.at[pl.ds(c * _AR + r0, _RT)])

    return pl.kernel(
        body,
        out_type=jax.ShapeDtypeStruct((2 * _AR, 128), jnp.float32),
        mesh=mesh, scratch_types=scratch)


def _stage_all(src2_h, sct2_h, cbase, sidx_all, sct_all):
    pltpu.sync_copy(src2_h.at[pl.ds(cbase, _NCP)], sidx_all)
    pltpu.sync_copy(sct2_h.at[pl.ds(cbase, _NCP)], sct_all)


def _make_gcn_kernel(C):
    ncol = C // 128
    mesh = plsc.VectorSubcoreMesh(core_axis_name="c", subcore_axis_name="s")
    scratch = [
        pltpu.VMEM((_NCP, 128), jnp.int32),
        pltpu.VMEM((_NCP, 128), jnp.int32),
        pltpu.VMEM((128, 128), jnp.float32),
        pltpu.VMEM((128, 128), jnp.float32),
        pltpu.VMEM_SHARED((_AR, 128), jnp.float32),
        pltpu.SemaphoreType.DMA,
        pltpu.SemaphoreType.DMA,
        pltpu.SemaphoreType.DMA,
        pltpu.SemaphoreType.DMA,
    ]

    def body(src2_h, sct2_h, tab_h, out_h,
             sidx_all, sct_all, rb0, rb1, acc, sg0, sg1, sw0, sw1):
        c = lax.axis_index("c")
        s = lax.axis_index("s")
        cbase = (c * 16 + s) * _NCP
        nch = _NCH
        r0 = s * _RT
        _stage_all(src2_h, sct2_h, cbase, sidx_all, sct_all)

        for k in range(ncol):
            _zero_acc(rb0, acc, r0)
            plsc.subcore_barrier()

            def g_hi(j):
                return tab_h.at[sidx_all.at[j, pl.ds(0, 64)],
                                pl.ds(k * 128, 128)]

            def g_lo(j):
                return tab_h.at[sidx_all.at[j, pl.ds(64, 64)],
                                pl.ds(k * 128, 128)]

            def fire_g(j, rb, sg):
                pltpu.async_copy(g_hi(j), rb.at[pl.ds(0, 64)], sg)
                pltpu.async_copy(g_lo(j), rb.at[pl.ds(64, 64)], sg)

            def wait_g(j, rb, sg):
                pltpu.make_async_copy(g_hi(j), rb.at[pl.ds(0, 64)], sg).wait()
                pltpu.make_async_copy(g_lo(j), rb.at[pl.ds(64, 64)], sg).wait()

            def w_dst(j):
                return acc.at[sct_all.at[j]]

            fire_g(0, rb0, sg0)
            fire_g(1, rb1, sg1)

            def pair(jj, _):
                j0 = jj * 2
                wait_g(j0, rb0, sg0)
                pltpu.async_copy(rb0, w_dst(j0), sw0, add=True)
                wait_g(j0 + 1, rb1, sg1)
                pltpu.async_copy(rb1, w_dst(j0 + 1), sw1, add=True)
                pltpu.make_async_copy(rb0, w_dst(j0), sw0).wait()
                fire_g(j0 + 2, rb0, sg0)
                pltpu.make_async_copy(rb1, w_dst(j0 + 1), sw1).wait()
                fire_g(j0 + 3, rb1, sg1)
                return 0
            lax.fori_loop(0, nch // 2 - 1, pair, 0)
            jl = nch - 2
            wait_g(jl, rb0, sg0)
            pltpu.async_copy(rb0, w_dst(jl), sw0, add=True)
            wait_g(jl + 1, rb1, sg1)
            pltpu.async_copy(rb1, w_dst(jl + 1), sw1, add=True)
            pltpu.make_async_copy(rb0, w_dst(jl), sw0).wait()
            pltpu.make_async_copy(rb1, w_dst(jl + 1), sw1).wait()

            plsc.subcore_barrier()
            pltpu.sync_copy(acc.at[pl.ds(r0, _RT)],
                            out_h.at[pl.ds(c * _AR + r0, _RT),
                                     pl.ds(k * 128, 128)])
            plsc.subcore_barrier()

    return pl.kernel(
        body,
        out_type=jax.ShapeDtypeStruct((2 * _AR, C), jnp.float32),
        mesh=mesh, scratch_types=scratch)


def _make_gat_kernel(C):
    ncol = C // 128
    mesh = plsc.VectorSubcoreMesh(core_axis_name="c", subcore_axis_name="s")
    scratch = [
        pltpu.VMEM((_NCP, 128), jnp.int32),
        pltpu.VMEM((_NCP, 128), jnp.int32),
        pltpu.VMEM((128,), jnp.int32),
        pltpu.VMEM((128, 128), jnp.float32),
        pltpu.VMEM((128, 128), jnp.float32),
        pltpu.VMEM((2048,), jnp.float32),
        pltpu.VMEM((2048,), jnp.float32),
        pltpu.VMEM_SHARED((_AR, 128), jnp.float32),
        pltpu.SemaphoreType.DMA,
        pltpu.SemaphoreType.DMA,
        pltpu.SemaphoreType.DMA,
        pltpu.SemaphoreType.DMA,
    ]

    def body(src2_h, sct2_h, tab_h, srep_h, drep_h, out_h, w_h,
             sidx_all, sct_all, didx, rb0, rb1, wb0, wb1, acc,
             sg0, sg1, sw0, sw1):
        c = lax.axis_index("c")
        s = lax.axis_index("s")
        cbase = (c * 16 + s) * _NCP
        nch = _NCH
        ebase = (c * 16 + s) * _SPAN
        r0 = s * _RT
        _stage_all(src2_h, sct2_h, cbase, sidx_all, sct_all)

        _zero_acc(rb0, acc, r0)
        plsc.subcore_barrier()

        def wsl(j):
            return w_h.at[pl.ds((ebase + j * 128) * _L, 128 * _L)]

        def chunk0(j, _):
            pltpu.async_copy(srep_h.at[sidx_all.at[j]], rb0, sg0)
            for kk in range(8):
                v = sct_all[j, pl.ds(kk * _L, _L)]
                didx[pl.ds(kk * _L, _L)] = jnp.where(v == _DUM, 0, v)
            pl.when(j > 0)(lambda: pltpu.make_async_copy(
                rb1, acc.at[sct_all.at[j - 1]], sw1).wait())
            pltpu.async_copy(drep_h.at[didx], rb1, sg1)
            pl.when(j > 0)(lambda: pltpu.make_async_copy(
                wb0, wsl(j - 1), sw0).wait())
            pltpu.make_async_copy(srep_h.at[sidx_all.at[j]], rb0, sg0).wait()
            pltpu.make_async_copy(drep_h.at[didx], rb1, sg1).wait()

            def wrow(i2, _):
                for u in range(2):
                    i = i2 * 2 + u
                    pv = rb0[i, pl.ds(0, _L)]
                    rv = rb0[i, pl.ds(_L, _L)]
                    qv = rb1[i, pl.ds(0, _L)]
                    tv = rb1[i, pl.ds(_L, _L)]
                    wv = jnp.maximum(pv * qv, rv * tv)
                    wb0[pl.ds(i * _L, _L)] = wv
                    for kk in range(8):
                        rb1[i, pl.ds(kk * _L, _L)] = wv
                return 0
            lax.fori_loop(0, 64, wrow, 0)
            pltpu.async_copy(wb0, wsl(j), sw0)
            pltpu.async_copy(rb1, acc.at[sct_all.at[j]], sw1, add=True)
            return 0
        lax.fori_loop(0, nch, chunk0, 0)
        pltpu.make_async_copy(rb1, acc.at[sct_all.at[nch - 1]], sw1).wait()
        pltpu.make_async_copy(wb0, wsl(nch - 1), sw0).wait()
        plsc.subcore_barrier()
        pltpu.sync_copy(acc.at[pl.ds(r0, _RT)],
                        out_h.at[pl.ds(c * _AR + r0, _RT),
                                 pl.ds(C, 128)])
        plsc.subcore_barrier()

        for k in range(ncol):
            _zero_acc(rb0, acc, r0)
            plsc.subcore_barrier()

            def g_ref(j):
                return tab_h.at[sidx_all.at[j], pl.ds(k * 128, 128)]

            def u_ref(j):
                return w_h.at[pl.ds((ebase + j * 128) * _L, 128 * _L)]

            def w_dst(j):
                return acc.at[sct_all.at[j]]

            def fire_g(j, rb, wb, sg):
                pltpu.async_copy(g_ref(j), rb, sg)
                pltpu.async_copy(u_ref(j), wb, sg)

            def wait_g(j, rb, wb, sg):
                pltpu.make_async_copy(g_ref(j), rb, sg).wait()
                pltpu.make_async_copy(u_ref(j), wb, sg).wait()

            def scale(rb, wb):
                def srow(i2, _):
                    for u in range(2):
                        i = i2 * 2 + u
                        wv = wb[pl.ds(i * _L, _L)]
                        for kk in range(8):
                            rb[i, pl.ds(kk * _L, _L)] = (
                                rb[i, pl.ds(kk * _L, _L)] * wv)
                    return 0
                lax.fori_loop(0, 64, srow, 0)

            fire_g(0, rb0, wb0, sg0)
            fire_g(1, rb1, wb1, sg1)

            def pair(jj, _):
                j0 = jj * 2
                wait_g(j0, rb0, wb0, sg0)
                scale(rb0, wb0)
                pltpu.async_copy(rb0, w_dst(j0), sw0, add=True)
                wait_g(j0 + 1, rb1, wb1, sg1)
                scale(rb1, wb1)
                pltpu.async_copy(rb1, w_dst(j0 + 1), sw1, add=True)
                pltpu.make_async_copy(rb0, w_dst(j0), sw0).wait()
                fire_g(j0 + 2, rb0, wb0, sg0)
                pltpu.make_async_copy(rb1, w_dst(j0 + 1), sw1).wait()
                fire_g(j0 + 3, rb1, wb1, sg1)
                return 0
            lax.fori_loop(0, nch // 2 - 1, pair, 0)
            jl = nch - 2
            wait_g(jl, rb0, wb0, sg0)
            scale(rb0, wb0)
            pltpu.async_copy(rb0, w_dst(jl), sw0, add=True)
            wait_g(jl + 1, rb1, wb1, sg1)
            scale(rb1, wb1)
            pltpu.async_copy(rb1, w_dst(jl + 1), sw1, add=True)
            pltpu.make_async_copy(rb0, w_dst(jl), sw0).wait()
            pltpu.make_async_copy(rb1, w_dst(jl + 1), sw1).wait()

            plsc.subcore_barrier()
            pltpu.sync_copy(acc.at[pl.ds(r0, _RT)],
                            out_h.at[pl.ds(c * _AR + r0, _RT),
                                     pl.ds(k * 128, 128)])
            plsc.subcore_barrier()

    return pl.kernel(
        body,
        out_type=(jax.ShapeDtypeStruct((2 * _AR, C + 128), jnp.float32),
                  jax.ShapeDtypeStruct((_EPAD * _L,), jnp.float32)),
        mesh=mesh, scratch_types=scratch)


def _splat(p, r):
    br = p.shape[0]
    return jnp.concatenate(
        [jnp.tile(p, (1, _L)), jnp.tile(r, (1, _L)),
         jnp.zeros((br, 128 - 2 * _L), jnp.float32)], axis=1)


def _t1_kern(x_ref, w_ref, dga_ref, dgb_ref, o_ref):
    deg = dga_ref[...][:, :1] + dgb_ref[...][:, :1]
    acc = jnp.dot(x_ref[...], w_ref[...], preferred_element_type=jnp.float32)
    o_ref[...] = acc * lax.rsqrt(deg)


def _t2_kern(ma_ref, mb_ref, dga_ref, dgb_ref, b_ref, w_ref, a_ref,
             h_ref, t_ref, s_ref, d_ref):
    deg = dga_ref[...][:, :1] + dgb_ref[...][:, :1]
    h = (ma_ref[...] + mb_ref[...]) * lax.rsqrt(deg) + b_ref[...]
    h_ref[...] = h
    xw = jnp.dot(h, w_ref[...], preferred_element_type=jnp.float32)
    t_ref[...] = xw
    sv = jnp.dot(xw, a_ref[...], preferred_element_type=jnp.float32)
    s1 = sv[:, 0:1]
    s2 = sv[:, 1:2]
    s_ref[...] = _splat(jnp.exp(s1), jnp.exp(0.2 * s1))
    d_ref[...] = _splat(jnp.exp(s2), jnp.exp(0.2 * s2))


def _t3_kern(h_ref, na_ref, nb_ref, batt_ref, g_ref, be_ref, wl_ref, bl_ref,
             watt_ref, a_ref, h2_ref, t_ref, s_ref, d_ref, *, d):
    nd = na_ref[...] + nb_ref[...]
    a = nd[:, :d] / nd[:, d:d + 1] + batt_ref[...]
    z = h_ref[...] + a
    mu = jnp.mean(z, axis=-1, keepdims=True)
    var = jnp.mean((z - mu) ** 2, axis=-1, keepdims=True)
    zn = (z - mu) * lax.rsqrt(var + 1e-5) * g_ref[...] + be_ref[...]
    h2 = jnp.maximum(
        jnp.dot(zn, wl_ref[...], preferred_element_type=jnp.float32)
        + bl_ref[...], 0.0)
    h2_ref[...] = h2
    xw = jnp.dot(h2, watt_ref[...], preferred_element_type=jnp.float32)
    t_ref[...] = xw
    sv = jnp.dot(xw, a_ref[...], preferred_element_type=jnp.float32)
    s1 = sv[:, 0:1]
    s2 = sv[:, 1:2]
    s_ref[...] = _splat(jnp.exp(s1), jnp.exp(0.2 * s1))
    d_ref[...] = _splat(jnp.exp(s2), jnp.exp(0.2 * s2))


def _t4_kern(h_ref, na_ref, nb_ref, batt_ref, g_ref, be_ref, wl_ref, bl_ref,
             wconv_ref, dga_ref, dgb_ref, t_ref, *, d):
    nd = na_ref[...] + nb_ref[...]
    a = nd[:, :d] / nd[:, d:d + 1] + batt_ref[...]
    z = h_ref[...] + a
    mu = jnp.mean(z, axis=-1, keepdims=True)
    var = jnp.mean((z - mu) ** 2, axis=-1, keepdims=True)
    zn = (z - mu) * lax.rsqrt(var + 1e-5) * g_ref[...] + be_ref[...]
    h3 = jnp.maximum(
        jnp.dot(zn, wl_ref[...], preferred_element_type=jnp.float32)
        + bl_ref[...], 0.0)
    deg = dga_ref[...][:, :1] + dgb_ref[...][:, :1]
    xw = jnp.dot(h3, wconv_ref[...], preferred_element_type=jnp.float32)
    br = xw.shape[0]
    t_ref[...] = jnp.concatenate(
        [xw * lax.rsqrt(deg), jnp.zeros((br, 64), jnp.float32)], axis=1)


def _t5_kern(ma_ref, mb_ref, dga_ref, dgb_ref, bconv_ref,
             w1_ref, b1_ref, w2_ref, b2_ref, o_ref):
    deg = dga_ref[...][:, :1] + dgb_ref[...][:, :1]
    m = ma_ref[...][:, :64] + mb_ref[...][:, :64]
    h = m * lax.rsqrt(deg) + bconv_ref[...]
    h = jnp.maximum(
        jnp.dot(h, w1_ref[...], preferred_element_type=jnp.float32)
        + b1_ref[...], 0.0)
    z = jnp.dot(h, w2_ref[...], preferred_element_type=jnp.float32) + b2_ref[...]
    lse = jax.scipy.special.logsumexp(z, axis=-1, keepdims=True)
    o_ref[...] = z - lse


def _rowA(c):
    return pl.BlockSpec((_BR, c), lambda i: (i, 0))


def _rowB(c):
    return pl.BlockSpec((_BR, c), lambda i: (i + _AR // _BR, 0))


def _full(r, c):
    return pl.BlockSpec((r, c), lambda i: (0, 0))


def _call_rows(kern, ins, in_specs, out_cols):
    out_specs = [_rowA(c) for c in out_cols]
    out_shape = [jax.ShapeDtypeStruct((_N, c), jnp.float32) for c in out_cols]
    if len(out_cols) == 1:
        out_specs, out_shape = out_specs[0], out_shape[0]
    return pl.pallas_call(
        kern, grid=(_NB,), in_specs=in_specs,
        out_specs=out_specs, out_shape=out_shape)(*ins)


def kernel(x, edge_index, params):
    p = params
    ar = jnp.arange(_N, dtype=edge_index.dtype)
    npadE = _EPAD - 17 * _N
    srcp = jnp.concatenate(
        [edge_index[0], ar, jnp.zeros((npadE,), jnp.int32)])
    dstp = jnp.concatenate(
        [edge_index[1], ar, jnp.full((npadE,), 1 << 29, jnp.int32)])
    sctp = jnp.where(dstp < _N, dstp, _DUM)
    srcp = jnp.pad(srcp.reshape(32, _NCH, 128),
                   ((0, 0), (0, _NCP - _NCH), (0, 0))).reshape(32 * _NCP, 128)
    sctp = jnp.pad(sctp.reshape(32, _NCH, 128),
                   ((0, 0), (0, _NCP - _NCH), (0, 0)),
                   constant_values=_DUM).reshape(32 * _NCP, 128)

    deg = _make_deg_kernel()(sctp)

    tab1 = _call_rows(
        _t1_kern, (x, p['W_emb'], deg, deg),
        [_rowA(256), _full(256, 512), _rowA(128), _rowB(128)], [512])
    m1 = _make_gcn_kernel(512)(srcp, sctp, tab1)

    a1 = jnp.stack([p['a1_src'], p['a1_dst']], axis=1)
    h, tab2, srep1, drep1 = _call_rows(
        _t2_kern,
        (m1, m1, deg, deg, p['b_emb'].reshape(1, -1), p['W_att1'], a1),
        [_rowA(512), _rowB(512), _rowA(128), _rowB(128),
         _full(1, 512), _full(512, 512), _full(512, 2)],
        [512, 512, 128, 128])
    nd1, _w1 = _make_gat_kernel(512)(srcp, sctp, tab2, srep1, drep1)

    a2 = jnp.stack([p['a2_src'], p['a2_dst']], axis=1)
    h2, tab3, srep2, drep2 = _call_rows(
        functools.partial(_t3_kern, d=512),
        (h, nd1, nd1, p['b_att1'].reshape(1, -1), p['g_n1'].reshape(1, -1),
         p['be_n1'].reshape(1, -1), p['W_l1'], p['b_l1'].reshape(1, -1),
         p['W_att2'], a2),
        [_rowA(512), _rowA(640), _rowB(640), _full(1, 512), _full(1, 512),
         _full(1, 512), _full(512, 256), _full(1, 256), _full(256, 256),
         _full(256, 2)],
        [256, 256, 128, 128])
    nd2, _w2 = _make_gat_kernel(256)(srcp, sctp, tab3, srep2, drep2)

    tab4 = _call_rows(
        functools.partial(_t4_kern, d=256),
        (h2, nd2, nd2, p['b_att2'].reshape(1, -1), p['g_n2'].reshape(1, -1),
         p['be_n2'].reshape(1, -1), p['W_l2'], p['b_l2'].reshape(1, -1),
         p['W_conv'], deg, deg),
        [_rowA(256), _rowA(384), _rowB(384), _full(1, 256), _full(1, 256),
         _full(1, 256), _full(256, 128), _full(1, 128), _full(128, 64),
         _rowA(128), _rowB(128)],
        [128])
    m3 = _make_gcn_kernel(128)(srcp, sctp, tab4)

    out = _call_rows(
        _t5_kern,
        (m3, m3, deg, deg, p['b_conv'].reshape(1, -1), p['W_c1'],
         p['b_c1'].reshape(1, -1), p['W_c2'], p['b_c2'].reshape(1, -1)),
        [_rowA(128), _rowB(128), _rowA(128), _rowB(128), _full(1, 64),
         _full(64, 32), _full(1, 32), _full(32, 10), _full(1, 10)],
        [10])
    return out
